# Initial kernel scaffold; baseline (speedup 1.0000x reference)
import jax
import jax.numpy as jnp
from jax import lax
from jax.experimental import pallas as pl
from jax.experimental.pallas import tpu as pltpu
from jax.experimental.pallas import tpu_sc as plsc

N = 10000
NP = 10240
E = 320000
NC = 2
NS = 16
NW = NC * NS
CH = 128
CHUNKS = -(-E // (NW * CH))
EP = NW * CHUNKS * CH
RPT = NP // NS

_f32 = jnp.float32


def _sc_mesh():
    return plsc.VectorSubcoreMesh(
        core_axis_name="c", subcore_axis_name="s", num_cores=NC, num_subcores=NS
    )


def _deg_body(dst_hbm, z_hbm, out_hbm, dst_v, ones_v, acc, sem):
    c = lax.axis_index("c")
    s = lax.axis_index("s")
    wid = c * NS + s
    r0 = s * RPT
    pltpu.sync_copy(z_hbm.at[pl.ds(r0, RPT)], acc.at[pl.ds(r0, RPT)])
    pltpu.sync_copy(dst_hbm.at[wid], dst_v)
    for i in range(CH // 16):
        ones_v[pl.ds(i * 16, 16)] = jnp.ones((16,), _f32)
    plsc.subcore_barrier()

    @pl.loop(0, CHUNKS)
    def _(j):
        pltpu.sync_copy(ones_v, acc.at[dst_v.at[j]], add=True)

    plsc.subcore_barrier()
    pltpu.sync_copy(acc.at[pl.ds(r0, RPT)], out_hbm.at[c, pl.ds(r0, RPT)])


def _deg_partials(dst_blk, zeros1):
    return pl.kernel(
        _deg_body,
        out_type=jax.ShapeDtypeStruct((NC, NP), _f32),
        mesh=_sc_mesh(),
        scratch_types=[
            pltpu.VMEM((CHUNKS, CH), jnp.int32),
            pltpu.VMEM((CH,), _f32),
            pltpu.VMEM_SHARED((NP,), _f32),
            pltpu.SemaphoreType.DMA,
        ],
    )(dst_blk, zeros1)


def _agg_body(src_hbm, dst_hbm, g_hbm, z_hbm, out_hbm, src_v, dst_v, rows_v, acc, sem):
    c = lax.axis_index("c")
    s = lax.axis_index("s")
    wid = c * NS + s
    r0 = s * RPT
    pltpu.sync_copy(z_hbm.at[pl.ds(r0, RPT)], acc.at[pl.ds(r0, RPT)])
    pltpu.sync_copy(src_hbm.at[wid], src_v)
    pltpu.sync_copy(dst_hbm.at[wid], dst_v)
    plsc.subcore_barrier()

    @pl.loop(0, CHUNKS)
    def _(j):
        pltpu.async_copy(g_hbm.at[src_v.at[j]], rows_v, sem).wait()
        pltpu.sync_copy(rows_v, acc.at[dst_v.at[j]], add=True)

    plsc.subcore_barrier()
    pltpu.sync_copy(acc.at[pl.ds(r0, RPT)], out_hbm.at[c, pl.ds(r0, RPT)])


def _agg_partials(src_blk, dst_blk, g, zeros_d, d):
    return pl.kernel(
        _agg_body,
        out_type=jax.ShapeDtypeStruct((NC, NP, d), _f32),
        mesh=_sc_mesh(),
        scratch_types=[
            pltpu.VMEM((CHUNKS, CH), jnp.int32),
            pltpu.VMEM((CHUNKS, CH), jnp.int32),
            pltpu.VMEM((CH, d), _f32),
            pltpu.VMEM_SHARED((NP, d), _f32),
            pltpu.SemaphoreType.DMA,
        ],
    )(src_blk, dst_blk, g, zeros_d)


_R = 1024


def _tca_body(deg0_ref, deg1_ref, x_ref, w_ref, g_ref, dinv_ref):
    i = pl.program_id(0)
    deg = deg0_ref[...] + deg1_ref[...] + 1.0
    rows = i * _R + lax.broadcasted_iota(jnp.int32, (_R, 1), 0)
    dinv = jnp.where(rows < N, lax.rsqrt(deg), 0.0)
    h = jnp.dot(x_ref[...], w_ref[...], preferred_element_type=_f32)
    g_ref[...] = h * dinv
    dinv_ref[...] = dinv


def _tc_g1(deg0, deg1, x, w1):
    ic = x.shape[1]
    hid = w1.shape[1]
    return pl.pallas_call(
        _tca_body,
        grid=(NP // _R,),
        in_specs=[
            pl.BlockSpec((_R, 1), lambda i: (i, 0)),
            pl.BlockSpec((_R, 1), lambda i: (i, 0)),
            pl.BlockSpec((_R, ic), lambda i: (i, 0)),
            pl.BlockSpec((ic, hid), lambda i: (0, 0)),
        ],
        out_specs=[
            pl.BlockSpec((_R, hid), lambda i: (i, 0)),
            pl.BlockSpec((_R, 1), lambda i: (i, 0)),
        ],
        out_shape=[
            jax.ShapeDtypeStruct((NP, hid), _f32),
            jax.ShapeDtypeStruct((NP, 1), _f32),
        ],
    )(deg0, deg1, x, w1)


def _tcb_body(s0_ref, s1_ref, g1_ref, dinv_ref, b1_ref, w2_ref, g2_ref):
    a = dinv_ref[...] * (s0_ref[...] + s1_ref[...] + g1_ref[...]) + b1_ref[...]
    a = jnp.maximum(a, 0.0)
    h2 = jnp.dot(a, w2_ref[...], preferred_element_type=_f32)
    g2_ref[...] = h2 * dinv_ref[...]


def _tc_g2(s0, s1, g1, dinv, b1, w2):
    hid = g1.shape[1]
    oc = w2.shape[1]
    return pl.pallas_call(
        _tcb_body,
        grid=(NP // _R,),
        in_specs=[
            pl.BlockSpec((_R, hid), lambda i: (i, 0)),
            pl.BlockSpec((_R, hid), lambda i: (i, 0)),
            pl.BlockSpec((_R, hid), lambda i: (i, 0)),
            pl.BlockSpec((_R, 1), lambda i: (i, 0)),
            pl.BlockSpec((1, hid), lambda i: (0, 0)),
            pl.BlockSpec((hid, oc), lambda i: (0, 0)),
        ],
        out_specs=pl.BlockSpec((_R, oc), lambda i: (i, 0)),
        out_shape=jax.ShapeDtypeStruct((NP, oc), _f32),
    )(s0, s1, g1, dinv, b1, w2)


def _tcc_body(s0_ref, s1_ref, g2_ref, dinv_ref, b2_ref, out_ref):
    out_ref[...] = (
        dinv_ref[...] * (s0_ref[...] + s1_ref[...] + g2_ref[...]) + b2_ref[...]
    )


def _tc_out(s0, s1, g2, dinv, b2):
    oc = g2.shape[1]
    return pl.pallas_call(
        _tcc_body,
        grid=(NP // _R,),
        in_specs=[
            pl.BlockSpec((_R, oc), lambda i: (i, 0)),
            pl.BlockSpec((_R, oc), lambda i: (i, 0)),
            pl.BlockSpec((_R, oc), lambda i: (i, 0)),
            pl.BlockSpec((_R, 1), lambda i: (i, 0)),
            pl.BlockSpec((1, oc), lambda i: (0, 0)),
        ],
        out_specs=pl.BlockSpec((_R, oc), lambda i: (i, 0)),
        out_shape=jax.ShapeDtypeStruct((NP, oc), _f32),
    )(s0, s1, g2, dinv, b2)


def kernel(x, edge_index, W1, b1, W2, b2):
    hid = W1.shape[1]
    oc = W2.shape[1]

    ei = edge_index.astype(jnp.int32)
    pad = EP - E
    fill = N + (jnp.arange(pad, dtype=jnp.int32) % (NP - N))
    src_blk = jnp.concatenate([ei[0], fill]).reshape(NW, CHUNKS, CH)
    dst_blk = jnp.concatenate([ei[1], fill]).reshape(NW, CHUNKS, CH)
    xp = jnp.pad(x, ((0, NP - N), (0, 0)))

    zeros1 = jnp.zeros((NP,), _f32)
    zeros_h = jnp.zeros((NP, hid), _f32)
    zeros_o = jnp.zeros((NP, oc), _f32)

    degp = _deg_partials(dst_blk, zeros1)
    deg0 = degp[0].reshape(NP, 1)
    deg1 = degp[1].reshape(NP, 1)

    g1, dinv = _tc_g1(deg0, deg1, xp, W1)
    s1 = _agg_partials(src_blk, dst_blk, g1, zeros_h, hid)
    g2 = _tc_g2(s1[0], s1[1], g1, dinv, b1.reshape(1, hid), W2)
    s2 = _agg_partials(src_blk, dst_blk, g2, zeros_o, oc)
    out = _tc_out(s2[0], s2[1], g2, dinv, b2.reshape(1, oc))
    return out[:N]


# speedup vs baseline: 23.7330x; 23.7330x over previous
import jax
import jax.numpy as jnp
from jax import lax
from jax.experimental import pallas as pl
from jax.experimental.pallas import tpu as pltpu
from jax.experimental.pallas import tpu_sc as plsc

N = 10000
NP = 10240
E = 320000
NC = 2
NS = 16
NW = NC * NS
CH = 128
CHUNKS = -(-E // (NW * CH))
EP = NW * CHUNKS * CH
RPT = NP // NS

_f32 = jnp.float32


def _sc_mesh():
    return plsc.VectorSubcoreMesh(
        core_axis_name="c", subcore_axis_name="s", num_cores=NC, num_subcores=NS
    )


def _deg_body(dst_hbm, z_hbm, out_hbm, dst_v, ones_v, acc, sem):
    c = lax.axis_index("c")
    s = lax.axis_index("s")
    wid = c * NS + s
    r0 = s * RPT
    pltpu.sync_copy(z_hbm.at[pl.ds(r0, RPT)], acc.at[pl.ds(r0, RPT)])
    pltpu.sync_copy(dst_hbm.at[wid], dst_v)
    for i in range(CH // 16):
        ones_v[pl.ds(i * 16, 16)] = jnp.ones((16,), _f32)
    plsc.subcore_barrier()

    @pl.loop(0, CHUNKS)
    def _(j):
        pltpu.sync_copy(ones_v, acc.at[dst_v.at[j]], add=True)

    plsc.subcore_barrier()
    pltpu.sync_copy(acc.at[pl.ds(r0, RPT)], out_hbm.at[c, pl.ds(r0, RPT)])


def _deg_partials(dst_blk, zeros1):
    return pl.kernel(
        _deg_body,
        out_type=jax.ShapeDtypeStruct((NC, NP), _f32),
        mesh=_sc_mesh(),
        scratch_types=[
            pltpu.VMEM((CHUNKS, CH), jnp.int32),
            pltpu.VMEM((CH,), _f32),
            pltpu.VMEM_SHARED((NP,), _f32),
            pltpu.SemaphoreType.DMA,
        ],
    )(dst_blk, zeros1)


def _agg_body(src_hbm, dst_hbm, g_hbm, z_hbm, out_hbm, src_v, dst_v, rows_v, acc, sem):
    c = lax.axis_index("c")
    s = lax.axis_index("s")
    wid = c * NS + s
    r0 = s * RPT
    pltpu.sync_copy(z_hbm.at[pl.ds(r0, RPT)], acc.at[pl.ds(r0, RPT)])
    pltpu.sync_copy(src_hbm.at[wid], src_v)
    pltpu.sync_copy(dst_hbm.at[wid], dst_v)
    plsc.subcore_barrier()

    @pl.loop(0, CHUNKS)
    def _(j):
        pltpu.async_copy(g_hbm.at[src_v.at[j]], rows_v, sem).wait()
        pltpu.sync_copy(rows_v, acc.at[dst_v.at[j]], add=True)

    plsc.subcore_barrier()
    pltpu.sync_copy(acc.at[pl.ds(r0, RPT)], out_hbm.at[c, pl.ds(r0, RPT)])


def _agg_partials(src_blk, dst_blk, g, zeros_d, d):
    return pl.kernel(
        _agg_body,
        out_type=jax.ShapeDtypeStruct((NC, NP, d), _f32),
        mesh=_sc_mesh(),
        scratch_types=[
            pltpu.VMEM((CHUNKS, CH), jnp.int32),
            pltpu.VMEM((CHUNKS, CH), jnp.int32),
            pltpu.VMEM((CH, d), _f32),
            pltpu.VMEM_SHARED((NP, d), _f32),
            pltpu.SemaphoreType.DMA,
        ],
        compiler_params=pltpu.CompilerParams(use_tc_tiling_on_sc=False),
    )(src_blk, dst_blk, g, zeros_d)


_R = 1024


def _tca_body(deg0_ref, deg1_ref, x_ref, w_ref, g_ref, dinv_ref):
    i = pl.program_id(0)
    deg = deg0_ref[...] + deg1_ref[...] + 1.0
    rows = i * _R + lax.broadcasted_iota(jnp.int32, (_R, 1), 0)
    dinv = jnp.where(rows < N, lax.rsqrt(deg), 0.0)
    h = jnp.dot(x_ref[...], w_ref[...], preferred_element_type=_f32)
    g_ref[...] = h * dinv
    dinv_ref[...] = dinv


def _tc_g1(deg0, deg1, x, w1):
    ic = x.shape[1]
    hid = w1.shape[1]
    return pl.pallas_call(
        _tca_body,
        grid=(NP // _R,),
        in_specs=[
            pl.BlockSpec((_R, 1), lambda i: (i, 0)),
            pl.BlockSpec((_R, 1), lambda i: (i, 0)),
            pl.BlockSpec((_R, ic), lambda i: (i, 0)),
            pl.BlockSpec((ic, hid), lambda i: (0, 0)),
        ],
        out_specs=[
            pl.BlockSpec((_R, hid), lambda i: (i, 0)),
            pl.BlockSpec((_R, 1), lambda i: (i, 0)),
        ],
        out_shape=[
            jax.ShapeDtypeStruct((NP, hid), _f32),
            jax.ShapeDtypeStruct((NP, 1), _f32),
        ],
    )(deg0, deg1, x, w1)


def _tcb_body(s0_ref, s1_ref, g1_ref, dinv_ref, b1_ref, w2_ref, g2_ref):
    a = dinv_ref[...] * (s0_ref[...] + s1_ref[...] + g1_ref[...]) + b1_ref[...]
    a = jnp.maximum(a, 0.0)
    h2 = jnp.dot(a, w2_ref[...], preferred_element_type=_f32)
    g2_ref[...] = h2 * dinv_ref[...]


def _tc_g2(s0, s1, g1, dinv, b1, w2):
    hid = g1.shape[1]
    oc = w2.shape[1]
    return pl.pallas_call(
        _tcb_body,
        grid=(NP // _R,),
        in_specs=[
            pl.BlockSpec((_R, hid), lambda i: (i, 0)),
            pl.BlockSpec((_R, hid), lambda i: (i, 0)),
            pl.BlockSpec((_R, hid), lambda i: (i, 0)),
            pl.BlockSpec((_R, 1), lambda i: (i, 0)),
            pl.BlockSpec((1, hid), lambda i: (0, 0)),
            pl.BlockSpec((hid, oc), lambda i: (0, 0)),
        ],
        out_specs=pl.BlockSpec((_R, oc), lambda i: (i, 0)),
        out_shape=jax.ShapeDtypeStruct((NP, oc), _f32),
    )(s0, s1, g1, dinv, b1, w2)


def _tcc_body(s0_ref, s1_ref, g2_ref, dinv_ref, b2_ref, out_ref):
    out_ref[...] = (
        dinv_ref[...] * (s0_ref[...] + s1_ref[...] + g2_ref[...]) + b2_ref[...]
    )


def _tc_out(s0, s1, g2, dinv, b2):
    oc = g2.shape[1]
    return pl.pallas_call(
        _tcc_body,
        grid=(NP // _R,),
        in_specs=[
            pl.BlockSpec((_R, oc), lambda i: (i, 0)),
            pl.BlockSpec((_R, oc), lambda i: (i, 0)),
            pl.BlockSpec((_R, oc), lambda i: (i, 0)),
            pl.BlockSpec((_R, 1), lambda i: (i, 0)),
            pl.BlockSpec((1, oc), lambda i: (0, 0)),
        ],
        out_specs=pl.BlockSpec((_R, oc), lambda i: (i, 0)),
        out_shape=jax.ShapeDtypeStruct((NP, oc), _f32),
    )(s0, s1, g2, dinv, b2)


def kernel(x, edge_index, W1, b1, W2, b2):
    hid = W1.shape[1]
    oc = W2.shape[1]

    ei = edge_index.astype(jnp.int32)
    pad = EP - E
    fill = N + (jnp.arange(pad, dtype=jnp.int32) % (NP - N))
    src_blk = jnp.concatenate([ei[0], fill]).reshape(NW, CHUNKS, CH)
    dst_blk = jnp.concatenate([ei[1], fill]).reshape(NW, CHUNKS, CH)
    xp = jnp.pad(x, ((0, NP - N), (0, 0)))

    zeros1 = jnp.zeros((NP,), _f32)
    zeros_h = jnp.zeros((NP, hid), _f32)
    zeros_o = jnp.zeros((NP, oc), _f32)

    degp = _deg_partials(dst_blk, zeros1)
    deg0 = degp[0].reshape(NP, 1)
    deg1 = degp[1].reshape(NP, 1)

    g1, dinv = _tc_g1(deg0, deg1, xp, W1)
    s1 = _agg_partials(src_blk, dst_blk, g1, zeros_h, hid)
    g2 = _tc_g2(s1[0], s1[1], g1, dinv, b1.reshape(1, hid), W2)
    s2 = _agg_partials(src_blk, dst_blk, g2, zeros_o, oc)
    out = _tc_out(s2[0], s2[1], g2, dinv, b2.reshape(1, oc))
    return out[:N]


# speedup vs baseline: 29.3795x; 1.2379x over previous
import jax
import jax.numpy as jnp
from jax import lax
from jax.experimental import pallas as pl
from jax.experimental.pallas import tpu as pltpu
from jax.experimental.pallas import tpu_sc as plsc

N = 10000
NP = 10240
E = 320000
NC = 2
NS = 16
NW = NC * NS
CH = 128
IGRP = 20
CHUNKS = 80
NIG = CHUNKS // IGRP
EP = NW * CHUNKS * CH
RPT = NP // NS

_f32 = jnp.float32


def _sc_mesh():
    return plsc.VectorSubcoreMesh(
        core_axis_name="c", subcore_axis_name="s", num_cores=NC, num_subcores=NS
    )


def _deg_body(dst_hbm, z_hbm, out_hbm, dst_v, ones_v, acc, sem):
    c = lax.axis_index("c")
    s = lax.axis_index("s")
    wid = c * NS + s
    r0 = s * RPT
    pltpu.sync_copy(z_hbm.at[pl.ds(r0, RPT)], acc.at[pl.ds(r0, RPT)])
    pltpu.sync_copy(dst_hbm.at[wid], dst_v)
    for i in range(CH // 16):
        ones_v[pl.ds(i * 16, 16)] = jnp.ones((16,), _f32)
    plsc.subcore_barrier()

    @pl.loop(0, CHUNKS)
    def _(j):
        pltpu.sync_copy(ones_v, acc.at[dst_v.at[j]], add=True)

    plsc.subcore_barrier()
    pltpu.sync_copy(acc.at[pl.ds(r0, RPT)], out_hbm.at[c, pl.ds(r0, RPT)])


def _deg_partials(dst_blk, zeros1):
    return pl.kernel(
        _deg_body,
        out_type=jax.ShapeDtypeStruct((NC, NP), _f32),
        mesh=_sc_mesh(),
        scratch_types=[
            pltpu.VMEM((CHUNKS, CH), jnp.int32),
            pltpu.VMEM((CH,), _f32),
            pltpu.VMEM_SHARED((NP,), _f32),
            pltpu.SemaphoreType.DMA,
        ],
    )(dst_blk, zeros1)


def _make_agg_body(nb):
    def _agg_body(sd_hbm, g_hbm, z_hbm, out_hbm, idx, rows, acc, isem, gsem, ssem):
        c = lax.axis_index("c")
        s = lax.axis_index("s")
        wid = c * NS + s
        r0 = s * RPT
        pltpu.sync_copy(z_hbm.at[pl.ds(r0, RPT)], acc.at[pl.ds(r0, RPT)])
        pltpu.sync_copy(sd_hbm.at[wid, 0], idx[0])
        plsc.subcore_barrier()

        for ig in range(NIG):
            p = ig % 2
            if ig + 1 < NIG:
                pltpu.async_copy(sd_hbm.at[wid, ig + 1], idx[1 - p], isem[1 - p])
            for b in range(nb):
                pltpu.async_copy(g_hbm.at[idx[p].at[0, b]], rows[b], gsem[b])

            @pl.loop(0, IGRP // nb - 1)
            def _(grp):
                base = grp * nb
                for b in range(nb):
                    pltpu.make_async_copy(g_hbm.at[idx[p].at[0, 0]], rows[b], gsem[b]).wait()
                    pltpu.async_copy(rows[b], acc.at[idx[p].at[1, base + b]], ssem[b], add=True)
                for b in range(nb):
                    pltpu.make_async_copy(rows[b], acc.at[idx[p].at[1, 0]], ssem[b]).wait()
                    pltpu.async_copy(g_hbm.at[idx[p].at[0, base + nb + b]], rows[b], gsem[b])

            last = IGRP - nb
            for b in range(nb):
                pltpu.make_async_copy(g_hbm.at[idx[p].at[0, 0]], rows[b], gsem[b]).wait()
                pltpu.async_copy(rows[b], acc.at[idx[p].at[1, last + b]], ssem[b], add=True)
            for b in range(nb):
                pltpu.make_async_copy(rows[b], acc.at[idx[p].at[1, 0]], ssem[b]).wait()
            if ig + 1 < NIG:
                pltpu.make_async_copy(sd_hbm.at[wid, 0], idx[1 - p], isem[1 - p]).wait()

        plsc.subcore_barrier()
        pltpu.sync_copy(acc.at[pl.ds(r0, RPT)], out_hbm.at[c, pl.ds(r0, RPT)])

    return _agg_body


def _agg_partials(sd_blk, g, zeros_d, d):
    nb = 2 if d >= 128 else 4
    return pl.kernel(
        _make_agg_body(nb),
        out_type=jax.ShapeDtypeStruct((NC, NP, d), _f32),
        mesh=_sc_mesh(),
        scratch_types=[
            [pltpu.VMEM((2, IGRP, CH), jnp.int32) for _ in range(2)],
            [pltpu.VMEM((CH, d), _f32) for _ in range(nb)],
            pltpu.VMEM_SHARED((NP, d), _f32),
            [pltpu.SemaphoreType.DMA for _ in range(2)],
            [pltpu.SemaphoreType.DMA for _ in range(nb)],
            [pltpu.SemaphoreType.DMA for _ in range(nb)],
        ],
        compiler_params=pltpu.CompilerParams(use_tc_tiling_on_sc=False),
    )(sd_blk, g, zeros_d)


_R = 1024


def _tca_body(deg0_ref, deg1_ref, x_ref, w_ref, g_ref, dinv_ref):
    i = pl.program_id(0)
    deg = deg0_ref[...] + deg1_ref[...] + 1.0
    rows = i * _R + lax.broadcasted_iota(jnp.int32, (_R, 1), 0)
    dinv = jnp.where(rows < N, lax.rsqrt(deg), 0.0)
    h = jnp.dot(x_ref[...], w_ref[...], preferred_element_type=_f32)
    g_ref[...] = h * dinv
    dinv_ref[...] = dinv


def _tc_g1(deg0, deg1, x, w1):
    ic = x.shape[1]
    hid = w1.shape[1]
    return pl.pallas_call(
        _tca_body,
        grid=(NP // _R,),
        in_specs=[
            pl.BlockSpec((_R, 1), lambda i: (i, 0)),
            pl.BlockSpec((_R, 1), lambda i: (i, 0)),
            pl.BlockSpec((_R, ic), lambda i: (i, 0)),
            pl.BlockSpec((ic, hid), lambda i: (0, 0)),
        ],
        out_specs=[
            pl.BlockSpec((_R, hid), lambda i: (i, 0)),
            pl.BlockSpec((_R, 1), lambda i: (i, 0)),
        ],
        out_shape=[
            jax.ShapeDtypeStruct((NP, hid), _f32),
            jax.ShapeDtypeStruct((NP, 1), _f32),
        ],
    )(deg0, deg1, x, w1)


def _tcb_body(s0_ref, s1_ref, g1_ref, dinv_ref, b1_ref, w2_ref, g2_ref):
    a = dinv_ref[...] * (s0_ref[...] + s1_ref[...] + g1_ref[...]) + b1_ref[...]
    a = jnp.maximum(a, 0.0)
    h2 = jnp.dot(a, w2_ref[...], preferred_element_type=_f32)
    g2_ref[...] = h2 * dinv_ref[...]


def _tc_g2(s0, s1, g1, dinv, b1, w2):
    hid = g1.shape[1]
    oc = w2.shape[1]
    return pl.pallas_call(
        _tcb_body,
        grid=(NP // _R,),
        in_specs=[
            pl.BlockSpec((_R, hid), lambda i: (i, 0)),
            pl.BlockSpec((_R, hid), lambda i: (i, 0)),
            pl.BlockSpec((_R, hid), lambda i: (i, 0)),
            pl.BlockSpec((_R, 1), lambda i: (i, 0)),
            pl.BlockSpec((1, hid), lambda i: (0, 0)),
            pl.BlockSpec((hid, oc), lambda i: (0, 0)),
        ],
        out_specs=pl.BlockSpec((_R, oc), lambda i: (i, 0)),
        out_shape=jax.ShapeDtypeStruct((NP, oc), _f32),
    )(s0, s1, g1, dinv, b1, w2)


def _tcc_body(s0_ref, s1_ref, g2_ref, dinv_ref, b2_ref, out_ref):
    out_ref[...] = (
        dinv_ref[...] * (s0_ref[...] + s1_ref[...] + g2_ref[...]) + b2_ref[...]
    )


def _tc_out(s0, s1, g2, dinv, b2):
    oc = g2.shape[1]
    return pl.pallas_call(
        _tcc_body,
        grid=(NP // _R,),
        in_specs=[
            pl.BlockSpec((_R, oc), lambda i: (i, 0)),
            pl.BlockSpec((_R, oc), lambda i: (i, 0)),
            pl.BlockSpec((_R, oc), lambda i: (i, 0)),
            pl.BlockSpec((_R, 1), lambda i: (i, 0)),
            pl.BlockSpec((1, oc), lambda i: (0, 0)),
        ],
        out_specs=pl.BlockSpec((_R, oc), lambda i: (i, 0)),
        out_shape=jax.ShapeDtypeStruct((NP, oc), _f32),
    )(s0, s1, g2, dinv, b2)


def kernel(x, edge_index, W1, b1, W2, b2):
    hid = W1.shape[1]
    oc = W2.shape[1]

    ei = edge_index.astype(jnp.int32)
    pad = EP - E
    fill = N + (jnp.arange(pad, dtype=jnp.int32) % (NP - N))
    srcp = jnp.concatenate([ei[0], fill])
    dstp = jnp.concatenate([ei[1], fill])
    dst_blk = dstp.reshape(NW, CHUNKS, CH)
    sd_blk = jnp.stack(
        [srcp.reshape(NW, NIG, IGRP, CH), dstp.reshape(NW, NIG, IGRP, CH)], axis=2
    )
    xp = jnp.pad(x, ((0, NP - N), (0, 0)))

    zeros1 = jnp.zeros((NP,), _f32)
    zeros_h = jnp.zeros((NP, hid), _f32)
    zeros_o = jnp.zeros((NP, oc), _f32)

    degp = _deg_partials(dst_blk, zeros1)
    deg0 = degp[0].reshape(NP, 1)
    deg1 = degp[1].reshape(NP, 1)

    g1, dinv = _tc_g1(deg0, deg1, xp, W1)
    s1 = _agg_partials(sd_blk, g1, zeros_h, hid)
    g2 = _tc_g2(s1[0], s1[1], g1, dinv, b1.reshape(1, hid), W2)
    s2 = _agg_partials(sd_blk, g2, zeros_o, oc)
    out = _tc_out(s2[0], s2[1], g2, dinv, b2.reshape(1, oc))
    return out[:N]
